# baseline (speedup 1.0000x reference)
import numpy as np
import jax
import jax.numpy as jnp
from jax import lax
from jax.experimental import pallas as pl
from jax.experimental.pallas import tpu as pltpu
from jax.experimental.pallas import tpu_sc as plsc

_SRC_PAD = 704
_OUT_PAD = 208
_LANES = 16


def _build_index_map():
    def g(src, dim, idx):
        dim = dim % src.ndim
        sl = tuple(
            slice(None) if a == dim else slice(0, idx.shape[a])
            for a in range(src.ndim)
        )
        return np.take_along_axis(src[sl], idx, axis=dim)

    bx = np.arange(12)
    by = np.arange(28).reshape(4, 7) + 12
    bz = np.arange(24).reshape(2, 3, 4) + 40
    bd = np.arange(625).reshape(5, 5, 5, 5) + 64

    ix = np.array([7, 9, 11])
    iy0 = np.array([[1, 3, 2], [0, 3, 1]])
    iy1 = np.array([[1, 3, 2, 4, 6, 5], [4, 3, 2, 1, 5, 6]])
    iz0 = np.array([[[0], [1], [0]], [[1], [0], [1]]])
    iz1 = np.array([[[0], [1], [2]], [[1], [2], [0]]])
    iz2 = np.array([[[0, 1, 2, 3]], [[2, 1, 0, 3]]])
    zz = np.array([[[[0, 1, 0, 1, 0], [1, 0, 1, 0, 1],
                     [0, 1, 0, 1, 0], [1, 0, 1, 0, 1]]],
                   [[[1, 0, 3, 4, 1], [0, 1, 0, 1, 0],
                     [1, 0, 1, 0, 1], [0, 1, 0, 1, 0]]]])

    parts = [
        g(bx, 0, ix),
        g(by, 0, iy0),
        g(by, 1, iy1),
        g(bz, -3, iz0),
        g(bz, -2, iz1),
        g(bz, -1, iz2),
        g(bd, 0, zz),
        g(bd, 1, zz),
        g(bd, 2, zz),
        g(bd, 3, zz),
    ]
    specs = []
    off = 0
    for p in parts:
        specs.append((off, p.shape))
        off += p.size
    flat = np.concatenate([p.ravel() for p in parts])
    flat = np.pad(flat, (0, _OUT_PAD - flat.size)).astype(np.int32)
    return flat, specs


_IDX_NP, _OUT_SPECS = _build_index_map()


def _gather_body(src_hbm, idx_hbm, out_hbm, idx_v, out_v, sem):
    wid = lax.axis_index("s") * 2 + lax.axis_index("c")

    @pl.when(wid == 0)
    def _():
        pltpu.sync_copy(idx_hbm, idx_v)
        pltpu.async_copy(src_hbm.at[idx_v], out_v, sem).wait()
        pltpu.sync_copy(out_v, out_hbm)


def kernel(x, y, z, d):
    flat = jnp.concatenate([x.ravel(), y.ravel(), z.ravel(), d.ravel()])
    flat = jnp.pad(flat, (0, _SRC_PAD - flat.shape[0]))
    idx = jnp.asarray(_IDX_NP)

    mesh = plsc.VectorSubcoreMesh(core_axis_name="c", subcore_axis_name="s")
    out_flat = pl.kernel(
        _gather_body,
        mesh=mesh,
        out_type=jax.ShapeDtypeStruct((_OUT_PAD,), jnp.float32),
        scratch_types=[
            pltpu.VMEM((_OUT_PAD,), jnp.int32),
            pltpu.VMEM((_OUT_PAD,), jnp.float32),
            pltpu.SemaphoreType.DMA,
        ],
    )(flat, idx)

    outs = []
    for off, shape in _OUT_SPECS:
        size = int(np.prod(shape))
        outs.append(out_flat[off:off + size].reshape(shape))
    return tuple(outs)
